# baseline (speedup 1.0000x reference)
import functools

import jax
import jax.numpy as jnp
from jax import lax
from jax.experimental import pallas as pl
from jax.experimental.pallas import tpu as pltpu
from jax.experimental.pallas import tpu_sc as plsc

SEQ = 10
LOC = 20
ED = 128
HALF = 64
BATCH = 16384

HALVES = BATCH * SEQ * 2
NC, NS = 2, 16
NW = NC * NS
HPW = HALVES // NW
CHUNK = 128
NCH = HPW // CHUNK


def _table_body(tok_ref, w_ref, b_ref, posh_ref, o_ref):
    t = jnp.dot(tok_ref[:], w_ref[:], preferred_element_type=jnp.float32)
    t = t + b_ref[:]
    o_ref[:] = posh_ref[:][:, None, :] + t[None, :, :]


def _build_table(tok_table, W, b, pos_table):
    return pl.pallas_call(
        _table_body,
        out_shape=jax.ShapeDtypeStruct((LOC, LOC, HALF), jnp.float32),
    )(tok_table, W, b.reshape(1, HALF), pos_table.reshape(2 * SEQ, HALF))


def _gather_body(g_hbm, v_hbm, out_hbm, vbuf, cidx, buf0, buf1, gsem0, gsem1):
    wid = lax.axis_index("s") * NC + lax.axis_index("c")
    kbase = wid * HPW

    pltpu.sync_copy(v_hbm.at[pl.ds(kbase, HPW)], vbuf)

    lanes = lax.iota(jnp.int32, 16)

    def idx_body(g, carry):
        v = vbuf[pl.ds(g * 16, 16)]
        m = (g * 16 + lanes) % (2 * SEQ)
        cidx[g // 8, pl.ds((g % 8) * 16, 16)] = m * LOC + v
        return carry

    lax.fori_loop(0, HPW // 16, idx_body, 0)

    def start_gather(t, buf, sem):
        pltpu.make_async_copy(g_hbm.at[cidx.at[t]], buf, sem).start()

    def wait_gather(buf, sem):
        pltpu.make_async_copy(g_hbm.at[cidx.at[0]], buf, sem).wait()

    def scatter(t, buf):
        pltpu.sync_copy(buf, out_hbm.at[pl.ds(kbase + t * CHUNK, CHUNK)])

    start_gather(0, buf0, gsem0)

    def pipe_body(t2, carry):
        t = t2 * 2
        start_gather(t + 1, buf1, gsem1)
        wait_gather(buf0, gsem0)
        scatter(t, buf0)

        @pl.when(t2 + 1 < NCH // 2)
        def _():
            start_gather(t + 2, buf0, gsem0)

        wait_gather(buf1, gsem1)
        scatter(t + 1, buf1)
        return carry

    lax.fori_loop(0, NCH // 2, pipe_body, 0)


def _gather_rows(g_flat, v_flat):
    mesh = plsc.VectorSubcoreMesh(core_axis_name="c", subcore_axis_name="s")
    f = functools.partial(
        pl.kernel,
        mesh=mesh,
        compiler_params=pltpu.CompilerParams(use_tc_tiling_on_sc=False),
        out_type=jax.ShapeDtypeStruct((HALVES, HALF), jnp.float32),
        scratch_types=[
            pltpu.VMEM((HPW,), jnp.int32),
            pltpu.VMEM((NCH, CHUNK), jnp.int32),
            pltpu.VMEM((CHUNK, HALF), jnp.float32),
            pltpu.VMEM((CHUNK, HALF), jnp.float32),
            pltpu.SemaphoreType.DMA,
            pltpu.SemaphoreType.DMA,
        ],
    )(_gather_body)
    return f(g_flat, v_flat)


def kernel(inputs, tok_table, W, b, pos_table):
    g = _build_table(tok_table, W, b, pos_table).reshape(LOC * LOC, HALF)
    v_flat = inputs.astype(jnp.int32).reshape(-1)
    out = _gather_rows(g, v_flat)
    return out.reshape(BATCH, SEQ, ED)


# speedup vs baseline: 5.1484x; 5.1484x over previous
import functools

import jax
import jax.numpy as jnp
from jax import lax
from jax.experimental import pallas as pl
from jax.experimental.pallas import tpu as pltpu
from jax.experimental.pallas import tpu_sc as plsc

SEQ = 10
LOC = 20
ED = 128
HALF = 64
BATCH = 16384

ROWS = BATCH * SEQ
NC, NS = 2, 16
NW = NC * NS
RPW = ROWS // NW
CHUNK = 128
NCH = RPW // CHUNK


def _table_body(tok_ref, w_ref, b_ref, pos_ref, o_ref):
    t = jnp.dot(tok_ref[:], w_ref[:], preferred_element_type=jnp.float32)
    t = t + b_ref[:]
    zeros = jnp.zeros((LOC, HALF), jnp.float32)
    tl = jnp.concatenate([t, zeros], axis=1)
    tr = jnp.concatenate([zeros, t], axis=1)
    g = (tl[None, :, None, :] + tr[None, None, :, :]
         + pos_ref[:][:, None, None, :])
    o_ref[:] = g


def _build_table(tok_table, W, b, pos_table):
    return pl.pallas_call(
        _table_body,
        out_shape=jax.ShapeDtypeStruct((SEQ, LOC, LOC, ED), jnp.float32),
    )(tok_table, W, b.reshape(1, HALF), pos_table)


def _gather_body(g_hbm, i0_hbm, i1_hbm, out_hbm, ibuf0, ibuf1, cidx, buf0,
                 buf1, gsem0, gsem1):
    wid = lax.axis_index("s") * NC + lax.axis_index("c")
    rowbase = wid * RPW

    pltpu.sync_copy(i0_hbm.at[pl.ds(rowbase, RPW)], ibuf0)
    pltpu.sync_copy(i1_hbm.at[pl.ds(rowbase, RPW)], ibuf1)

    def idx_body(g, carry):
        i0 = ibuf0[pl.ds(g * 16, 16)]
        i1 = ibuf1[pl.ds(g * 16, 16)]
        s = (rowbase + g * 16) // BATCH
        c = s * (LOC * LOC) + i0 * LOC + i1
        cidx[g // 8, pl.ds((g % 8) * 16, 16)] = c
        return carry

    lax.fori_loop(0, RPW // 16, idx_body, 0)

    def start_gather(t, buf, sem):
        pltpu.make_async_copy(g_hbm.at[cidx.at[t]], buf, sem).start()

    def wait_gather(buf, sem):
        pltpu.make_async_copy(g_hbm.at[cidx.at[0]], buf, sem).wait()

    def scatter(t, buf):
        pltpu.sync_copy(buf, out_hbm.at[pl.ds(rowbase + t * CHUNK, CHUNK)])

    start_gather(0, buf0, gsem0)

    def pipe_body(t2, carry):
        t = t2 * 2
        start_gather(t + 1, buf1, gsem1)
        wait_gather(buf0, gsem0)
        scatter(t, buf0)

        @pl.when(t2 + 1 < NCH // 2)
        def _():
            start_gather(t + 2, buf0, gsem0)

        wait_gather(buf1, gsem1)
        scatter(t + 1, buf1)
        return carry

    lax.fori_loop(0, NCH // 2, pipe_body, 0)


def _gather_rows(g_flat, i0_flat, i1_flat):
    mesh = plsc.VectorSubcoreMesh(core_axis_name="c", subcore_axis_name="s")
    f = functools.partial(
        pl.kernel,
        mesh=mesh,
        out_type=jax.ShapeDtypeStruct((ROWS, ED), jnp.float32),
        scratch_types=[
            pltpu.VMEM((RPW,), jnp.int32),
            pltpu.VMEM((RPW,), jnp.int32),
            pltpu.VMEM((NCH, CHUNK), jnp.int32),
            pltpu.VMEM((CHUNK, ED), jnp.float32),
            pltpu.VMEM((CHUNK, ED), jnp.float32),
            pltpu.SemaphoreType.DMA,
            pltpu.SemaphoreType.DMA,
        ],
    )(_gather_body)
    return f(g_flat, i0_flat, i1_flat)


def kernel(inputs, tok_table, W, b, pos_table):
    g = _build_table(tok_table, W, b, pos_table).reshape(SEQ * LOC * LOC, ED)
    idx = jnp.transpose(inputs.astype(jnp.int32), (1, 0, 2))
    i0_flat = idx[:, :, 0].reshape(-1)
    i1_flat = idx[:, :, 1].reshape(-1)
    out = _gather_rows(g, i0_flat, i1_flat)
    return jnp.transpose(out.reshape(SEQ, BATCH, ED), (1, 0, 2))


# speedup vs baseline: 5.4836x; 1.0651x over previous
import functools

import jax
import jax.numpy as jnp
from jax import lax
from jax.experimental import pallas as pl
from jax.experimental.pallas import tpu as pltpu
from jax.experimental.pallas import tpu_sc as plsc

SEQ = 10
LOC = 20
ED = 128
HALF = 64
BATCH = 16384

ROWS = BATCH * SEQ
NC, NS = 2, 16
NW = NC * NS
RPW = ROWS // NW
CHUNK = 128
NCH = RPW // CHUNK


def _table_body(tok_ref, w_ref, b_ref, pos_ref, o_ref):
    t = jnp.dot(tok_ref[:], w_ref[:], preferred_element_type=jnp.float32)
    t = t + b_ref[:]
    zeros = jnp.zeros((LOC, HALF), jnp.float32)
    tl = jnp.concatenate([t, zeros], axis=1)
    tr = jnp.concatenate([zeros, t], axis=1)
    g = (tl[None, :, None, :] + tr[None, None, :, :]
         + pos_ref[:][:, None, None, :])
    o_ref[:] = g


def _build_table(tok_table, W, b, pos_table):
    return pl.pallas_call(
        _table_body,
        out_shape=jax.ShapeDtypeStruct((SEQ, LOC, LOC, ED), jnp.float32),
    )(tok_table, W, b.reshape(1, HALF), pos_table)


def _gather_body(g_hbm, q_hbm, out_hbm, qbuf, cidx, buf0,
                 buf1, gsem0, gsem1):
    wid = lax.axis_index("s") * NC + lax.axis_index("c")
    rowbase = wid * RPW

    pltpu.sync_copy(q_hbm.at[pl.ds(rowbase * 2, RPW * 2)], qbuf)

    def idx_body(g, carry):
        base = (g // 8) * 256 + (g % 8) * 16
        i0 = qbuf[pl.ds(base, 16)]
        i1 = qbuf[pl.ds(base + 128, 16)]
        s = (rowbase + g * 16) // BATCH
        c = s * (LOC * LOC) + i0 * LOC + i1
        cidx[g // 8, pl.ds((g % 8) * 16, 16)] = c
        return carry

    lax.fori_loop(0, RPW // 16, idx_body, 0)

    def start_gather(t, buf, sem):
        pltpu.make_async_copy(g_hbm.at[cidx.at[t]], buf, sem).start()

    def wait_gather(buf, sem):
        pltpu.make_async_copy(g_hbm.at[cidx.at[0]], buf, sem).wait()

    def scatter(t, buf):
        pltpu.sync_copy(buf, out_hbm.at[pl.ds(rowbase + t * CHUNK, CHUNK)])

    start_gather(0, buf0, gsem0)

    def pipe_body(t2, carry):
        t = t2 * 2
        start_gather(t + 1, buf1, gsem1)
        wait_gather(buf0, gsem0)
        scatter(t, buf0)

        @pl.when(t2 + 1 < NCH // 2)
        def _():
            start_gather(t + 2, buf0, gsem0)

        wait_gather(buf1, gsem1)
        scatter(t + 1, buf1)
        return carry

    lax.fori_loop(0, NCH // 2, pipe_body, 0)


def _gather_rows(g_flat, q_flat):
    mesh = plsc.VectorSubcoreMesh(core_axis_name="c", subcore_axis_name="s")
    f = functools.partial(
        pl.kernel,
        mesh=mesh,
        out_type=jax.ShapeDtypeStruct((ROWS, ED), jnp.float32),
        scratch_types=[
            pltpu.VMEM((2 * RPW,), jnp.int32),
            pltpu.VMEM((NCH, CHUNK), jnp.int32),
            pltpu.VMEM((CHUNK, ED), jnp.float32),
            pltpu.VMEM((CHUNK, ED), jnp.float32),
            pltpu.SemaphoreType.DMA,
            pltpu.SemaphoreType.DMA,
        ],
    )(_gather_body)
    return f(g_flat, q_flat)


def kernel(inputs, tok_table, W, b, pos_table):
    g = _build_table(tok_table, W, b, pos_table).reshape(SEQ * LOC * LOC, ED)
    q = jnp.transpose(
        inputs.astype(jnp.int32).reshape(BATCH // 128, 128, SEQ, 2),
        (2, 0, 3, 1),
    ).reshape(-1)
    out = _gather_rows(g, q)
    return jnp.transpose(out.reshape(SEQ, BATCH, ED), (1, 0, 2))
